# baseline (speedup 1.0000x reference)
import functools

import jax
import jax.numpy as jnp
from jax import lax
from jax.experimental import pallas as pl
from jax.experimental.pallas import tpu as pltpu
from jax.experimental.pallas import tpu_sc as plsc

_NC, _NS = 2, 16
_NW = _NC * _NS
_E = 320000
_BPW = _E // _NW
_CH = 80
_NCHUNK = _BPW // _CH


def _sc_gather2_body(ti_hbm, tj_hbm, dst_hbm, src_hbm, g1_hbm, g2_hbm,
                     idx_d_v, idx_s_v, rows1_v, rows2_v, sem1, sem2):
    wid = lax.axis_index("s") * _NC + lax.axis_index("c")
    base0 = wid * _BPW
    pltpu.sync_copy(dst_hbm.at[wid], idx_d_v)
    pltpu.sync_copy(src_hbm.at[wid], idx_s_v)

    def body(i, carry):
        base = base0 + i * _CH
        cp1 = pltpu.async_copy(ti_hbm.at[idx_d_v.at[i]], rows1_v, sem1)
        cp2 = pltpu.async_copy(tj_hbm.at[idx_s_v.at[i]], rows2_v, sem2)
        cp1.wait()
        cp2.wait()
        pltpu.sync_copy(rows1_v, g1_hbm.at[pl.ds(base, _CH)])
        pltpu.sync_copy(rows2_v, g2_hbm.at[pl.ds(base, _CH)])
        return carry

    lax.fori_loop(0, _NCHUNK, body, 0)


_NP = 10240
_TSL = _NP // _NS


def _sc_denom_body(logits_hbm, dst_hbm, gmax_hbm, ex_hbm, denom_hbm,
                   logit_v, ex_v, idx_v, gmax_v, zbuf_v, den_sh):
    sid = lax.axis_index("s")
    cid = lax.axis_index("c")
    wid = sid * _NC + cid
    z16 = jnp.zeros((16,), jnp.float32)
    for g in range(_TSL // 16):
        zbuf_v[pl.ds(g * 16, 16)] = z16
    pltpu.sync_copy(zbuf_v, den_sh.at[pl.ds(sid * _TSL, _TSL)])
    pltpu.sync_copy(logits_hbm.at[wid], logit_v)
    pltpu.sync_copy(dst_hbm.at[wid], idx_v)
    pltpu.sync_copy(gmax_hbm, gmax_v)
    gmax = gmax_v[...]
    plsc.subcore_barrier()

    def body(i, carry):
        for g in range(_CH // 16):
            lv = logit_v[i, pl.ds(g * 16, 16)]
            ex_v[i, pl.ds(g * 16, 16)] = jnp.exp(lv - gmax)
        pltpu.sync_copy(ex_v.at[i], den_sh.at[idx_v.at[i]], add=True)
        return carry

    lax.fori_loop(0, _NCHUNK, body, 0)
    pltpu.sync_copy(ex_v, ex_hbm.at[wid])
    plsc.subcore_barrier()
    pltpu.sync_copy(den_sh.at[pl.ds(sid * _TSL, _TSL)],
                    denom_hbm.at[cid, pl.ds(sid * _TSL, _TSL)])


def _sc_denom(logits3, dst3, gmax16):
    mesh = plsc.VectorSubcoreMesh(core_axis_name="c", subcore_axis_name="s")
    f = pl.kernel(
        _sc_denom_body,
        mesh=mesh,
        out_type=[
            jax.ShapeDtypeStruct((_NW, _NCHUNK, _CH), jnp.float32),
            jax.ShapeDtypeStruct((_NC, _NP), jnp.float32),
        ],
        scratch_types=[
            pltpu.VMEM((_NCHUNK, _CH), jnp.float32),
            pltpu.VMEM((_NCHUNK, _CH), jnp.float32),
            pltpu.VMEM((_NCHUNK, _CH), jnp.int32),
            pltpu.VMEM((16,), jnp.float32),
            pltpu.VMEM((_TSL,), jnp.float32),
            pltpu.VMEM_SHARED((_NP,), jnp.float32),
        ],
    )
    return f(logits3, dst3, gmax16)


def _sc_gather2(table_i, table_j, dst, src):
    h = table_i.shape[1]
    dst3 = dst.reshape(_NW, _NCHUNK, _CH)
    src3 = src.reshape(_NW, _NCHUNK, _CH)
    mesh = plsc.VectorSubcoreMesh(core_axis_name="c", subcore_axis_name="s")
    f = pl.kernel(
        _sc_gather2_body,
        mesh=mesh,
        out_type=[
            jax.ShapeDtypeStruct((_E, h), jnp.float32),
            jax.ShapeDtypeStruct((_E, h), jnp.float32),
        ],
        scratch_types=[
            pltpu.VMEM((_NCHUNK, _CH), jnp.int32),
            pltpu.VMEM((_NCHUNK, _CH), jnp.int32),
            pltpu.VMEM((_CH, h), jnp.float32),
            pltpu.VMEM((_CH, h), jnp.float32),
            pltpu.SemaphoreType.DMA,
            pltpu.SemaphoreType.DMA,
        ],
    )
    return f(table_i, table_j, dst3, src3)


def _final_proj_kernel(h_ref, wc_ref, bc_ref, out_ref):
    out_ref[...] = h_ref[...] @ wc_ref[...] + bc_ref[0]


def _final_proj(h, Wc, bc):
    n = h.shape[0]
    blk = 2000
    return pl.pallas_call(
        _final_proj_kernel,
        grid=(n // blk,),
        in_specs=[
            pl.BlockSpec((blk, 128), lambda i: (i, 0)),
            pl.BlockSpec((128, 1), lambda i: (0, 0)),
            pl.BlockSpec(memory_space=pltpu.SMEM),
        ],
        out_specs=pl.BlockSpec((blk, 1), lambda i: (i, 0)),
        out_shape=jax.ShapeDtypeStruct((n, 1), jnp.float32),
    )(h, Wc, bc)


def _layer(x, src, dst, c, Wn, Wi, Wj, av, n, We_next):
    xWi = x @ Wi
    xWj = x @ Wj
    xWn = x @ Wn
    g1, g2 = _sc_gather2(xWi, xWj, dst, src)
    f = g1 + g2 + c
    e_act = jnp.where(f > 0, f, 0.2 * f)
    logits = e_act @ av
    gmax = jnp.max(logits)
    logits3 = logits.reshape(_NW, _NCHUNK, _CH)
    dst3 = dst.reshape(_NW, _NCHUNK, _CH)
    gmax16 = jnp.full((16,), gmax, jnp.float32)
    ex3, denom2 = _sc_denom(logits3, dst3, gmax16)
    ex = ex3.reshape(_E)
    denom = denom2[0] + denom2[1]
    alpha = ex / (denom[dst] + 1e-16)
    msg = alpha[:, None] * xWn[src]
    out = jnp.zeros((n, xWn.shape[1]), jnp.float32).at[dst].add(msg)
    c_next = f @ We_next if We_next is not None else None
    return out, c_next


def kernel(x, edge_index, edge_attr, Wn1, Wi1, Wj1, We1, av1, Wn2, Wi2, Wj2, We2, av2, Wn3, Wi3, Wj3, We3, av3, Wc, bc):
    n = x.shape[0]
    src = edge_index[0]
    dst = edge_index[1]
    c1 = edge_attr @ We1
    h, c2 = _layer(x, src, dst, c1, Wn1, Wi1, Wj1, av1, n, We2)
    h = jax.nn.relu(h)
    h, c3 = _layer(h, src, dst, c2, Wn2, Wi2, Wj2, av2, n, We3)
    h = jax.nn.relu(h)
    h, _ = _layer(h, src, dst, c3, Wn3, Wi3, Wj3, av3, n, None)
    h = jax.nn.relu(h)
    return _final_proj(h, Wc, bc)


# speedup vs baseline: 2.4344x; 2.4344x over previous
import functools

import jax
import jax.numpy as jnp
from jax import lax
from jax.experimental import pallas as pl
from jax.experimental.pallas import tpu as pltpu
from jax.experimental.pallas import tpu_sc as plsc

_NC, _NS = 2, 16
_NW = _NC * _NS
_E = 320000
_BPW = _E // _NW
_CH = 80
_NCHUNK = _BPW // _CH


def _sc_gather2_body(ti_hbm, tj_hbm, dst_hbm, src_hbm, g1_hbm, g2_hbm,
                     idx_d_v, idx_s_v, rows1_v, rows2_v, sem1, sem2):
    wid = lax.axis_index("s") * _NC + lax.axis_index("c")
    base0 = wid * _BPW
    pltpu.sync_copy(dst_hbm.at[wid], idx_d_v)
    pltpu.sync_copy(src_hbm.at[wid], idx_s_v)

    def body(i, carry):
        base = base0 + i * _CH
        cp1 = pltpu.async_copy(ti_hbm.at[idx_d_v.at[i]], rows1_v, sem1)
        cp2 = pltpu.async_copy(tj_hbm.at[idx_s_v.at[i]], rows2_v, sem2)
        cp1.wait()
        cp2.wait()
        pltpu.sync_copy(rows1_v, g1_hbm.at[pl.ds(base, _CH)])
        pltpu.sync_copy(rows2_v, g2_hbm.at[pl.ds(base, _CH)])
        return carry

    lax.fori_loop(0, _NCHUNK, body, 0)


_NP = 10240
_TSL = _NP // _NS


def _sc_denom_body(logits_hbm, dst_hbm, gmax_hbm, ex_hbm, denom_hbm,
                   logit_v, ex_v, idx_v, gmax_v, zbuf_v, den_sh):
    sid = lax.axis_index("s")
    cid = lax.axis_index("c")
    wid = sid * _NC + cid
    z16 = jnp.zeros((16,), jnp.float32)
    for g in range(_TSL // 16):
        zbuf_v[pl.ds(g * 16, 16)] = z16
    pltpu.sync_copy(zbuf_v, den_sh.at[pl.ds(sid * _TSL, _TSL)])
    pltpu.sync_copy(logits_hbm.at[wid], logit_v)
    pltpu.sync_copy(dst_hbm.at[wid], idx_v)
    pltpu.sync_copy(gmax_hbm, gmax_v)
    gmax = gmax_v[...]
    plsc.subcore_barrier()

    def body(i, carry):
        for g in range(_CH // 16):
            lv = logit_v[i, pl.ds(g * 16, 16)]
            ex_v[i, pl.ds(g * 16, 16)] = jnp.exp(lv - gmax)
        pltpu.sync_copy(ex_v.at[i], den_sh.at[idx_v.at[i]], add=True)
        return carry

    lax.fori_loop(0, _NCHUNK, body, 0)
    pltpu.sync_copy(ex_v, ex_hbm.at[wid])
    plsc.subcore_barrier()
    pltpu.sync_copy(den_sh.at[pl.ds(sid * _TSL, _TSL)],
                    denom_hbm.at[cid, pl.ds(sid * _TSL, _TSL)])


def _sc_denom(logits3, dst3, gmax16):
    mesh = plsc.VectorSubcoreMesh(core_axis_name="c", subcore_axis_name="s")
    f = pl.kernel(
        _sc_denom_body,
        mesh=mesh,
        out_type=[
            jax.ShapeDtypeStruct((_NW, _NCHUNK, _CH), jnp.float32),
            jax.ShapeDtypeStruct((_NC, _NP), jnp.float32),
        ],
        scratch_types=[
            pltpu.VMEM((_NCHUNK, _CH), jnp.float32),
            pltpu.VMEM((_NCHUNK, _CH), jnp.float32),
            pltpu.VMEM((_NCHUNK, _CH), jnp.int32),
            pltpu.VMEM((16,), jnp.float32),
            pltpu.VMEM((_TSL,), jnp.float32),
            pltpu.VMEM_SHARED((_NP,), jnp.float32),
        ],
    )
    return f(logits3, dst3, gmax16)


_NP2 = _NP // 2
_TSL2 = _NP2 // _NS
_NCHUNK2 = _E // _NS // _CH


_NPQ = _NP // 4
_TSLQ = _NPQ // _NS


def _sc_msg_body(tn_hbm, src_hbm, dst_hbm, ex_hbm, denom_hbm, out_hbm,
                 idx_s_v, idx_d_v, idx_c_v, ex_v, alpha_v, d_v, rows_v,
                 acc_sh, sem, sem2):
    sid = lax.axis_index("s")
    cid = lax.axis_index("c")
    z16 = jnp.zeros((16,), jnp.float32)
    z16i = jnp.zeros((16,), jnp.int32)
    zf16 = jnp.zeros((16,), jnp.float32)
    pltpu.sync_copy(src_hbm.at[sid], idx_s_v)
    pltpu.sync_copy(dst_hbm.at[sid], idx_d_v)
    pltpu.sync_copy(ex_hbm.at[sid], ex_v)

    for p in range(2):
        for r in range(_CH):
            for g in range(8):
                rows_v[r, pl.ds(g * 16, 16)] = z16
        for b in range(_TSLQ // _CH):
            pltpu.sync_copy(rows_v, acc_sh.at[pl.ds(sid * _TSLQ + b * _CH, _CH)])
        plsc.subcore_barrier()
        lo = cid * _NP2 + p * _NPQ

        def body(i, carry):
            cp1 = pltpu.async_copy(tn_hbm.at[idx_s_v.at[i]], rows_v, sem)
            cp2 = pltpu.async_copy(denom_hbm.at[idx_d_v.at[i]], d_v, sem2)
            cp1.wait()
            cp2.wait()
            for g in range(_CH // 16):
                d16 = d_v[pl.ds(g * 16, 16)]
                ex16 = ex_v[i, pl.ds(g * 16, 16)]
                dv = idx_d_v[i, pl.ds(g * 16, 16)] - lo
                inb = (dv >= 0) & (dv < _NPQ)
                idx_c_v[pl.ds(g * 16, 16)] = jnp.where(inb, dv, z16i)
                alpha_v[pl.ds(g * 16, 16)] = jnp.where(inb, ex16 / d16, zf16)
            for g16 in range(_CH // 16):
                av16 = alpha_v[pl.ds(g16 * 16, 16)]
                for j in range(16):
                    r = g16 * 16 + j
                    ar = av16[j]
                    for g in range(8):
                        rows_v[r, pl.ds(g * 16, 16)] = (
                            rows_v[r, pl.ds(g * 16, 16)] * ar)
            pltpu.sync_copy(rows_v, acc_sh.at[idx_c_v], add=True)
            return carry

        lax.fori_loop(0, _NCHUNK2, body, 0)
        plsc.subcore_barrier()
        pltpu.sync_copy(acc_sh.at[pl.ds(sid * _TSLQ, _TSLQ)],
                        out_hbm.at[cid * 2 + p, pl.ds(sid * _TSLQ, _TSLQ)])
        plsc.subcore_barrier()


def _sc_msg(table_n, src2, dst2, ex2, denom):
    h = table_n.shape[1]
    mesh = plsc.VectorSubcoreMesh(core_axis_name="c", subcore_axis_name="s")
    f = pl.kernel(
        _sc_msg_body,
        mesh=mesh,
        out_type=jax.ShapeDtypeStruct((4, _NPQ, h), jnp.float32),
        scratch_types=[
            pltpu.VMEM((_NCHUNK2, _CH), jnp.int32),
            pltpu.VMEM((_NCHUNK2, _CH), jnp.int32),
            pltpu.VMEM((_CH,), jnp.int32),
            pltpu.VMEM((_NCHUNK2, _CH), jnp.float32),
            pltpu.VMEM((_CH,), jnp.float32),
            pltpu.VMEM((_CH,), jnp.float32),
            pltpu.VMEM((_CH, h), jnp.float32),
            pltpu.VMEM_SHARED((_NPQ, h), jnp.float32),
            pltpu.SemaphoreType.DMA,
            pltpu.SemaphoreType.DMA,
        ],
    )
    return f(table_n, src2, dst2, ex2, denom)


def _sc_gather2(table_i, table_j, dst, src):
    h = table_i.shape[1]
    dst3 = dst.reshape(_NW, _NCHUNK, _CH)
    src3 = src.reshape(_NW, _NCHUNK, _CH)
    mesh = plsc.VectorSubcoreMesh(core_axis_name="c", subcore_axis_name="s")
    f = pl.kernel(
        _sc_gather2_body,
        mesh=mesh,
        out_type=[
            jax.ShapeDtypeStruct((_E, h), jnp.float32),
            jax.ShapeDtypeStruct((_E, h), jnp.float32),
        ],
        scratch_types=[
            pltpu.VMEM((_NCHUNK, _CH), jnp.int32),
            pltpu.VMEM((_NCHUNK, _CH), jnp.int32),
            pltpu.VMEM((_CH, h), jnp.float32),
            pltpu.VMEM((_CH, h), jnp.float32),
            pltpu.SemaphoreType.DMA,
            pltpu.SemaphoreType.DMA,
        ],
    )
    return f(table_i, table_j, dst3, src3)


def _final_proj_kernel(h_ref, wc_ref, bc_ref, out_ref):
    out_ref[...] = h_ref[...] @ wc_ref[...] + bc_ref[0]


def _final_proj(h, Wc, bc):
    n = h.shape[0]
    blk = 2000
    return pl.pallas_call(
        _final_proj_kernel,
        grid=(n // blk,),
        in_specs=[
            pl.BlockSpec((blk, 128), lambda i: (i, 0)),
            pl.BlockSpec((128, 1), lambda i: (0, 0)),
            pl.BlockSpec(memory_space=pltpu.SMEM),
        ],
        out_specs=pl.BlockSpec((blk, 1), lambda i: (i, 0)),
        out_shape=jax.ShapeDtypeStruct((n, 1), jnp.float32),
    )(h, Wc, bc)


def _layer(x, src, dst, c, Wn, Wi, Wj, av, n, We_next):
    xWi = x @ Wi
    xWj = x @ Wj
    xWn = x @ Wn
    g1, g2 = _sc_gather2(xWi, xWj, dst, src)
    f = g1 + g2 + c
    e_act = jnp.where(f > 0, f, 0.2 * f)
    logits = e_act @ av
    gmax = jnp.max(logits)
    logits3 = logits.reshape(_NW, _NCHUNK, _CH)
    dst3 = dst.reshape(_NW, _NCHUNK, _CH)
    gmax16 = jnp.full((16,), gmax, jnp.float32)
    ex3, denom2 = _sc_denom(logits3, dst3, gmax16)
    denom = denom2[0] + denom2[1] + 1e-16
    src2 = src.reshape(_NS, _NCHUNK2, _CH)
    dst2 = dst.reshape(_NS, _NCHUNK2, _CH)
    ex2 = ex3.reshape(_NS, _NCHUNK2, _CH)
    out4 = _sc_msg(xWn, src2, dst2, ex2, denom)
    out = out4.reshape(4 * _NPQ, -1)[:n]
    c_next = f @ We_next if We_next is not None else None
    return out, c_next


def kernel(x, edge_index, edge_attr, Wn1, Wi1, Wj1, We1, av1, Wn2, Wi2, Wj2, We2, av2, Wn3, Wi3, Wj3, We3, av3, Wc, bc):
    n = x.shape[0]
    src = edge_index[0]
    dst = edge_index[1]
    c1 = edge_attr @ We1
    h, c2 = _layer(x, src, dst, c1, Wn1, Wi1, Wj1, av1, n, We2)
    h = jax.nn.relu(h)
    h, c3 = _layer(h, src, dst, c2, Wn2, Wi2, Wj2, av2, n, We3)
    h = jax.nn.relu(h)
    h, _ = _layer(h, src, dst, c3, Wn3, Wi3, Wj3, av3, n, None)
    h = jax.nn.relu(h)
    return _final_proj(h, Wc, bc)
